# baseline (speedup 1.0000x reference)
import functools

import jax
import jax.numpy as jnp
from jax import lax
from jax.experimental import pallas as pl
from jax.experimental.pallas import tpu as pltpu
from jax.experimental.pallas import tpu_sc as plsc

_B = 16384
_K = 20
_D = 64
_NC = 2
_NS = 16
_NW = _NC * _NS
_CPW = _B // _NW
_CH = 64
_NCHUNK = _CPW // _CH
_VPW = 1000000 // _NW
_TBLK = 500
_FPW = _B * _K // _NW // 128

_MESH = plsc.VectorSubcoreMesh(core_axis_name="c", subcore_axis_name="s")


def _worker_id():
    return lax.axis_index("s") * _NC + lax.axis_index("c")


def _make_depad_kernel():
    @functools.partial(
        pl.kernel,
        mesh=_MESH,
        compiler_params=pltpu.CompilerParams(needs_layout_passes=False,
                                             use_tc_tiling_on_sc=True),
        out_type=[jax.ShapeDtypeStruct((_B * _K // 128, 128), jnp.int32)],
        scratch_types=[
            pltpu.VMEM((_CPW, _K), jnp.int32),
            pltpu.VMEM((_FPW, 128), jnp.int32),
        ],
    )
    def depad_body(neg_hbm, out_hbm, n_in, n_out):
        wid = _worker_id()
        lane = lax.iota(jnp.int32, 16)
        pltpu.sync_copy(neg_hbm.at[pl.ds(wid * _CPW, _CPW), :], n_in)

        def row_body(j, carry):
            for c8 in range(8):
                p = j * 128 + c8 * 16 + lane
                r = p // _K
                cc = p - r * _K
                n_out[j, pl.ds(c8 * 16, 16)] = plsc.load_gather(n_in, [r, cc])
            return carry

        lax.fori_loop(0, _FPW, row_body, 0)
        pltpu.sync_copy(n_out, out_hbm.at[pl.ds(wid * _FPW, _FPW)])

    return depad_body


def _make_repad_kernel():
    @functools.partial(
        pl.kernel,
        mesh=_MESH,
        compiler_params=pltpu.CompilerParams(needs_layout_passes=False,
                                             use_tc_tiling_on_sc=True),
        out_type=[jax.ShapeDtypeStruct((_B, _K), jnp.float32)],
        scratch_types=[
            pltpu.VMEM((_FPW, 128), jnp.float32),
            pltpu.VMEM((_CPW, _K), jnp.float32),
        ],
    )
    def repad_body(in_hbm, out_hbm, s_in, s_out):
        wid = _worker_id()
        lane = lax.iota(jnp.int32, 16)
        tail = lane < (_K - 16)
        pmax = _FPW * 128 - 1
        pltpu.sync_copy(in_hbm.at[pl.ds(wid * _FPW, _FPW)], s_in)

        def row_body(s, carry):
            p = s * _K + lane
            r = p // 128
            s_out[s, pl.ds(0, 16)] = plsc.load_gather(s_in, [r, p - r * 128])
            p2 = jnp.minimum(p + 16, pmax)
            r2 = p2 // 128
            plsc.store_scatter(s_out,
                               [jnp.full((16,), s, jnp.int32), 16 + lane],
                               plsc.load_gather(s_in, [r2, p2 - r2 * 128]),
                               mask=tail)
            return carry

        lax.fori_loop(0, _CPW, row_body, 0)
        pltpu.sync_copy(s_out, out_hbm.at[pl.ds(wid * _CPW, _CPW), :])

    return repad_body


def _make_transpose_kernel():
    bw = 896
    nfull = 1116

    @functools.partial(
        pl.kernel,
        mesh=_MESH,
        compiler_params=pltpu.CompilerParams(needs_layout_passes=False,
                                             use_tc_tiling_on_sc=True),
        out_type=[jax.ShapeDtypeStruct((1000000 * _D,), jnp.float32)],
        scratch_types=[
            pltpu.VMEM((_D, bw), jnp.float32),
            pltpu.VMEM((bw * _D,), jnp.float32),
            pltpu.VMEM((_D, 64), jnp.float32),
        ],
    )
    def tr_body(twt_hbm, out_hbm, in_v, out_v, tail_v):
        wid = _worker_id()
        lane = lax.iota(jnp.int32, 16)

        perms = [lax.rem(lane + k, jnp.full((16,), 16, jnp.int32))
                 for k in range(16)]

        def transpose_block(nvt, src):
            def qloop(q, c2):
                for d0 in range(0, _D, 16):
                    dl = d0 + lane
                    for k in range(16):
                        col = q * 16 + perms[k]
                        g = plsc.load_gather(src, [dl, col])
                        plsc.store_scatter(out_v, [col * _D + dl], g)
                return c2
            lax.fori_loop(0, nvt, qloop, 0)

        def blk(i, carry):
            b = wid + i * _NW

            @pl.when(b < nfull)
            def _():
                pltpu.sync_copy(twt_hbm.at[:, pl.ds(b * bw, bw)], in_v)
                transpose_block(bw // 16, in_v)
                pltpu.sync_copy(out_v, out_hbm.at[pl.ds(b * bw * _D,
                                                        bw * _D)])
            return carry

        lax.fori_loop(0, (nfull + _NW - 1) // _NW, blk, 0)

        @pl.when(wid == 0)
        def _():
            pltpu.sync_copy(twt_hbm.at[:, pl.ds(nfull * bw, 64)], tail_v)
            transpose_block(4, tail_v)
            pltpu.sync_copy(out_v.at[pl.ds(0, 64 * _D)],
                            out_hbm.at[pl.ds(nfull * bw * _D, 64 * _D)])

    return tr_body


def _make_main_kernel():
    @functools.partial(
        pl.kernel,
        mesh=_MESH,
        compiler_params=pltpu.CompilerParams(needs_layout_passes=False,
                                             use_tc_tiling_on_sc=False),
        out_type=[
            jax.ShapeDtypeStruct((_B,), jnp.float32),
            jax.ShapeDtypeStruct((_B * _K,), jnp.float32),
        ],
        scratch_types=[
            pltpu.VMEM((_CPW,), jnp.int32),
            pltpu.VMEM((_CPW,), jnp.int32),
            pltpu.VMEM((_FPW, 128), jnp.int32),
            pltpu.VMEM((_CH * _D,), jnp.int32),
            pltpu.VMEM((_CH * _D,), jnp.float32),
            pltpu.VMEM((_CH, _D), jnp.float32),
            pltpu.VMEM((_CH * _K, _D), jnp.float32),
            pltpu.VMEM((_CH,), jnp.float32),
            pltpu.VMEM((_CH * _K,), jnp.float32),
            pltpu.SemaphoreType.DMA,
        ],
    )
    def sc_body(tgt_hbm, ctx_hbm, neg_hbm, tw_hbm, cw_hbm,
                pos_hbm, nsc_hbm,
                t_idx_v, c_idx_v, n_idx_v, t_gidx, t_rows, c_rows, n_rows,
                pos_buf, neg_buf, sem):
        wid = _worker_id()
        lane = lax.iota(jnp.int32, 16)
        dcol = [(i * 16 + lane) * _CH for i in range(_D // 16)]

        wbase = wid * _CPW
        pltpu.sync_copy(tgt_hbm.at[pl.ds(wbase, _CPW)], t_idx_v)
        pltpu.sync_copy(ctx_hbm.at[pl.ds(wbase, _CPW)], c_idx_v)
        pltpu.sync_copy(neg_hbm.at[pl.ds(wid * _FPW, _FPW)], n_idx_v)

        nrows_per_chunk = _CH * _K // 128

        def chunk_body(ci, carry):
            base = wbase + ci * _CH

            tslice = [t_idx_v[pl.ds(ci * _CH + g * 16, 16)]
                      for g in range(_CH // 16)]

            def t_idx_body(d, carry2):
                dm = d * 1000000
                for g in range(_CH // 16):
                    t_gidx[pl.ds(d * _CH + g * 16, 16)] = dm + tslice[g]
                return carry2

            lax.fori_loop(0, _D, t_idx_body, 0)

            def t_fire(m, carry2):
                pltpu.async_copy(
                    tw_hbm.at[t_gidx.at[pl.ds(m * 128, 128)]],
                    t_rows.at[pl.ds(m * 128, 128)], sem)
                return carry2

            lax.fori_loop(0, _CH * _D // 128, t_fire, 0)

            cps = [
                pltpu.async_copy(cw_hbm.at[c_idx_v.at[pl.ds(ci * _CH, _CH)]],
                                 c_rows, sem),
            ]
            for j in range(nrows_per_chunk):
                cps.append(
                    pltpu.async_copy(
                        cw_hbm.at[n_idx_v.at[ci * nrows_per_chunk + j]],
                        n_rows.at[pl.ds(j * 128, 128)],
                        sem,
                    )
                )
            for cp in cps:
                cp.wait()

            def t_drain(m, carry2):
                pltpu.make_async_copy(
                    tw_hbm.at[t_gidx.at[pl.ds(m * 128, 128)]],
                    t_rows.at[pl.ds(m * 128, 128)], sem).wait()
                return carry2

            lax.fori_loop(0, _CH * _D // 128, t_drain, 0)

            last = lane == 15

            def s_body(s, carry2):
                tv = [plsc.load_gather(t_rows, [dcol[i] + s])
                      for i in range(_D // 16)]
                cv = [c_rows[s, pl.ds(i * 16, 16)] for i in range(_D // 16)]
                acc = tv[0] * cv[0]
                for i in range(1, _D // 16):
                    acc = acc + tv[i] * cv[i]
                plsc.store_scatter(pos_buf, [jnp.full((16,), s, jnp.int32)],
                                   plsc.cumsum(acc), mask=last)
                for k in range(_K):
                    r = s * _K + k
                    nacc = tv[0] * n_rows[r, pl.ds(0, 16)]
                    for i in range(1, _D // 16):
                        nacc = nacc + tv[i] * n_rows[r, pl.ds(i * 16, 16)]
                    plsc.store_scatter(neg_buf,
                                       [jnp.full((16,), r, jnp.int32)],
                                       -plsc.cumsum(nacc), mask=last)
                return carry2

            lax.fori_loop(0, _CH, s_body, 0)

            pltpu.sync_copy(pos_buf, pos_hbm.at[pl.ds(base, _CH)])
            pltpu.sync_copy(neg_buf, nsc_hbm.at[pl.ds(base * _K, _CH * _K)])
            return carry

        lax.fori_loop(0, _NCHUNK, chunk_body, 0)

    return sc_body


_DEPAD = _make_depad_kernel()
_REPAD = _make_repad_kernel()
_TRANS = _make_transpose_kernel()
_MAIN = _make_main_kernel()


def kernel(target, context, negatives, target_W, context_W):
    t = target.astype(jnp.int32)
    c = context.astype(jnp.int32)
    (n2,) = _DEPAD(negatives.astype(jnp.int32))
    (cwf,) = _TRANS(context_W.T)
    pos, neg1d = _MAIN(t, c, n2,
                       target_W.T.reshape(-1), cwf.reshape(1000000, _D))
    (neg,) = _REPAD(neg1d.reshape(_B * _K // 128, 128))
    return pos, neg


# speedup vs baseline: 4.5077x; 4.5077x over previous
import functools

import jax
import jax.numpy as jnp
from jax import lax
from jax.experimental import pallas as pl
from jax.experimental.pallas import tpu as pltpu
from jax.experimental.pallas import tpu_sc as plsc

_B = 16384
_K = 20
_D = 64
_NC = 2
_NS = 16
_NW = _NC * _NS
_CPW = _B // _NW
_CH = 64
_NCHUNK = _CPW // _CH
_VPW = 1000000 // _NW
_TBLK = 500
_FPW = _B * _K // _NW // 128

_MESH = plsc.VectorSubcoreMesh(core_axis_name="c", subcore_axis_name="s")


def _worker_id():
    return lax.axis_index("s") * _NC + lax.axis_index("c")


def _make_depad_kernel():
    @functools.partial(
        pl.kernel,
        mesh=_MESH,
        compiler_params=pltpu.CompilerParams(needs_layout_passes=False,
                                             use_tc_tiling_on_sc=True),
        out_type=[jax.ShapeDtypeStruct((_B * _K // 128, 128), jnp.int32)],
        scratch_types=[
            pltpu.VMEM((_CPW, _K), jnp.int32),
            pltpu.VMEM((_FPW, 128), jnp.int32),
        ],
    )
    def depad_body(neg_hbm, out_hbm, n_in, n_out):
        wid = _worker_id()
        lane = lax.iota(jnp.int32, 16)
        pltpu.sync_copy(neg_hbm.at[pl.ds(wid * _CPW, _CPW), :], n_in)

        def row_body(j, carry):
            for c8 in range(8):
                p = j * 128 + c8 * 16 + lane
                r = p // _K
                cc = p - r * _K
                n_out[j, pl.ds(c8 * 16, 16)] = plsc.load_gather(n_in, [r, cc])
            return carry

        lax.fori_loop(0, _FPW, row_body, 0)
        pltpu.sync_copy(n_out, out_hbm.at[pl.ds(wid * _FPW, _FPW)])

    return depad_body


def _make_repad_kernel():
    @functools.partial(
        pl.kernel,
        mesh=_MESH,
        compiler_params=pltpu.CompilerParams(needs_layout_passes=False,
                                             use_tc_tiling_on_sc=True),
        out_type=[jax.ShapeDtypeStruct((_B, _K), jnp.float32)],
        scratch_types=[
            pltpu.VMEM((_FPW, 128), jnp.float32),
            pltpu.VMEM((_CPW, _K), jnp.float32),
        ],
    )
    def repad_body(in_hbm, out_hbm, s_in, s_out):
        wid = _worker_id()
        lane = lax.iota(jnp.int32, 16)
        tail = lane < (_K - 16)
        pmax = _FPW * 128 - 1
        pltpu.sync_copy(in_hbm.at[pl.ds(wid * _FPW, _FPW)], s_in)

        def row_body(s, carry):
            p = s * _K + lane
            r = p // 128
            s_out[s, pl.ds(0, 16)] = plsc.load_gather(s_in, [r, p - r * 128])
            p2 = jnp.minimum(p + 16, pmax)
            r2 = p2 // 128
            plsc.store_scatter(s_out,
                               [jnp.full((16,), s, jnp.int32), 16 + lane],
                               plsc.load_gather(s_in, [r2, p2 - r2 * 128]),
                               mask=tail)
            return carry

        lax.fori_loop(0, _CPW, row_body, 0)
        pltpu.sync_copy(s_out, out_hbm.at[pl.ds(wid * _CPW, _CPW), :])

    return repad_body


def _make_transpose_kernel():
    bw = 896
    nfull = 1116

    @functools.partial(
        pl.kernel,
        mesh=_MESH,
        compiler_params=pltpu.CompilerParams(needs_layout_passes=False,
                                             use_tc_tiling_on_sc=True),
        out_type=[jax.ShapeDtypeStruct((1000000 * _D,), jnp.float32)],
        scratch_types=[
            pltpu.VMEM((_D, bw), jnp.float32),
            pltpu.VMEM((bw * _D,), jnp.float32),
            pltpu.VMEM((_D, 64), jnp.float32),
        ],
    )
    def tr_body(twt_hbm, out_hbm, in_v, out_v, tail_v):
        wid = _worker_id()
        lane = lax.iota(jnp.int32, 16)

        perms = [lax.rem(lane + k, jnp.full((16,), 16, jnp.int32))
                 for k in range(16)]

        def transpose_block(nvt, src):
            def qloop(q, c2):
                for d0 in range(0, _D, 16):
                    dl = d0 + lane
                    for k in range(16):
                        col = q * 16 + perms[k]
                        g = plsc.load_gather(src, [dl, col])
                        plsc.store_scatter(out_v, [col * _D + dl], g)
                return c2
            lax.fori_loop(0, nvt, qloop, 0)

        def blk(i, carry):
            b = wid + i * _NW

            @pl.when(b < nfull)
            def _():
                pltpu.sync_copy(twt_hbm.at[:, pl.ds(b * bw, bw)], in_v)
                transpose_block(bw // 16, in_v)
                pltpu.sync_copy(out_v, out_hbm.at[pl.ds(b * bw * _D,
                                                        bw * _D)])
            return carry

        lax.fori_loop(0, (nfull + _NW - 1) // _NW, blk, 0)

        @pl.when(wid == 0)
        def _():
            pltpu.sync_copy(twt_hbm.at[:, pl.ds(nfull * bw, 64)], tail_v)
            transpose_block(4, tail_v)
            pltpu.sync_copy(out_v.at[pl.ds(0, 64 * _D)],
                            out_hbm.at[pl.ds(nfull * bw * _D, 64 * _D)])

    return tr_body


def _make_main_kernel():
    @functools.partial(
        pl.kernel,
        mesh=_MESH,
        compiler_params=pltpu.CompilerParams(needs_layout_passes=False,
                                             use_tc_tiling_on_sc=False),
        out_type=[
            jax.ShapeDtypeStruct((_B,), jnp.float32),
            jax.ShapeDtypeStruct((_B * _K,), jnp.float32),
        ],
        scratch_types=[
            pltpu.VMEM((_CPW,), jnp.int32),
            pltpu.VMEM((_CPW,), jnp.int32),
            pltpu.VMEM((_FPW, 128), jnp.int32),
            pltpu.VMEM((_CH, _D), jnp.float32),
            pltpu.VMEM((_CH, _D), jnp.float32),
            pltpu.VMEM((_CH * _K, _D), jnp.float32),
            pltpu.VMEM((_CH,), jnp.float32),
            pltpu.VMEM((_CH * _K,), jnp.float32),
            pltpu.SemaphoreType.DMA,
        ],
    )
    def sc_body(tgt_hbm, ctx_hbm, neg_hbm, tw_hbm, cw_hbm,
                pos_hbm, nsc_hbm,
                t_idx_v, c_idx_v, n_idx_v, t_rows, c_rows, n_rows,
                pos_buf, neg_buf, sem):
        wid = _worker_id()
        lane = lax.iota(jnp.int32, 16)

        wbase = wid * _CPW
        pltpu.sync_copy(tgt_hbm.at[pl.ds(wbase, _CPW)], t_idx_v)
        pltpu.sync_copy(ctx_hbm.at[pl.ds(wbase, _CPW)], c_idx_v)
        pltpu.sync_copy(neg_hbm.at[pl.ds(wid * _FPW, _FPW)], n_idx_v)

        nrows_per_chunk = _CH * _K // 128

        def chunk_body(ci, carry):
            base = wbase + ci * _CH

            cps = [
                pltpu.async_copy(tw_hbm.at[t_idx_v.at[pl.ds(ci * _CH, _CH)]],
                                 t_rows, sem),
                pltpu.async_copy(cw_hbm.at[c_idx_v.at[pl.ds(ci * _CH, _CH)]],
                                 c_rows, sem),
            ]
            for j in range(nrows_per_chunk):
                cps.append(
                    pltpu.async_copy(
                        cw_hbm.at[n_idx_v.at[ci * nrows_per_chunk + j]],
                        n_rows.at[pl.ds(j * 128, 128)],
                        sem,
                    )
                )
            for cp in cps:
                cp.wait()

            last = lane == 15

            def s_body(s, carry2):
                tv = [t_rows[s, pl.ds(i * 16, 16)] for i in range(_D // 16)]
                cv = [c_rows[s, pl.ds(i * 16, 16)] for i in range(_D // 16)]
                acc = tv[0] * cv[0]
                for i in range(1, _D // 16):
                    acc = acc + tv[i] * cv[i]
                plsc.store_scatter(pos_buf, [jnp.full((16,), s, jnp.int32)],
                                   plsc.cumsum(acc), mask=last)
                for k in range(_K):
                    r = s * _K + k
                    nacc = tv[0] * n_rows[r, pl.ds(0, 16)]
                    for i in range(1, _D // 16):
                        nacc = nacc + tv[i] * n_rows[r, pl.ds(i * 16, 16)]
                    plsc.store_scatter(neg_buf,
                                       [jnp.full((16,), r, jnp.int32)],
                                       -plsc.cumsum(nacc), mask=last)
                return carry2

            lax.fori_loop(0, _CH, s_body, 0)

            pltpu.sync_copy(pos_buf, pos_hbm.at[pl.ds(base, _CH)])
            pltpu.sync_copy(neg_buf, nsc_hbm.at[pl.ds(base * _K, _CH * _K)])
            return carry

        lax.fori_loop(0, _NCHUNK, chunk_body, 0)

    return sc_body


_DEPAD = _make_depad_kernel()
_REPAD = _make_repad_kernel()
_TRANS = _make_transpose_kernel()
_MAIN = _make_main_kernel()


def kernel(target, context, negatives, target_W, context_W):
    t = target.astype(jnp.int32)
    c = context.astype(jnp.int32)
    (n2,) = _DEPAD(negatives.astype(jnp.int32))
    (cwf,) = _TRANS(context_W.T)
    pos, neg1d = _MAIN(t, c, n2,
                       target_W, cwf.reshape(1000000, _D))
    (neg,) = _REPAD(neg1d.reshape(_B * _K // 128, 128))
    return pos, neg


# speedup vs baseline: 4.5099x; 1.0005x over previous
import functools

import jax
import jax.numpy as jnp
from jax import lax
from jax.experimental import pallas as pl
from jax.experimental.pallas import tpu as pltpu
from jax.experimental.pallas import tpu_sc as plsc

_B = 16384
_K = 20
_D = 64
_NC = 2
_NS = 16
_NW = _NC * _NS
_CPW = _B // _NW
_CH = 64
_NCHUNK = _CPW // _CH
_FPW = _B * _K // _NW // 128

_MESH = plsc.VectorSubcoreMesh(core_axis_name="c", subcore_axis_name="s")


def _worker_id():
    return lax.axis_index("s") * _NC + lax.axis_index("c")


def _make_depad_kernel():
    @functools.partial(
        pl.kernel,
        mesh=_MESH,
        compiler_params=pltpu.CompilerParams(needs_layout_passes=False,
                                             use_tc_tiling_on_sc=True),
        out_type=[jax.ShapeDtypeStruct((_B * _K // 128, 128), jnp.int32)],
        scratch_types=[
            pltpu.VMEM((_CPW, _K), jnp.int32),
            pltpu.VMEM((_FPW, 128), jnp.int32),
        ],
    )
    def depad_body(neg_hbm, out_hbm, n_in, n_out):
        wid = _worker_id()
        lane = lax.iota(jnp.int32, 16)
        pltpu.sync_copy(neg_hbm.at[pl.ds(wid * _CPW, _CPW), :], n_in)

        def row_body(j, carry):
            for c8 in range(8):
                p = j * 128 + c8 * 16 + lane
                r = p // _K
                cc = p - r * _K
                n_out[j, pl.ds(c8 * 16, 16)] = plsc.load_gather(n_in, [r, cc])
            return carry

        lax.fori_loop(0, _FPW, row_body, 0)
        pltpu.sync_copy(n_out, out_hbm.at[pl.ds(wid * _FPW, _FPW)])

    return depad_body


def _make_repad_kernel():
    @functools.partial(
        pl.kernel,
        mesh=_MESH,
        compiler_params=pltpu.CompilerParams(needs_layout_passes=False,
                                             use_tc_tiling_on_sc=True),
        out_type=[jax.ShapeDtypeStruct((_B, _K), jnp.float32)],
        scratch_types=[
            pltpu.VMEM((_FPW, 128), jnp.float32),
            pltpu.VMEM((_CPW, _K), jnp.float32),
        ],
    )
    def repad_body(in_hbm, out_hbm, s_in, s_out):
        wid = _worker_id()
        lane = lax.iota(jnp.int32, 16)
        tail = lane < (_K - 16)
        pmax = _FPW * 128 - 1
        pltpu.sync_copy(in_hbm.at[pl.ds(wid * _FPW, _FPW)], s_in)

        def row_body(s, carry):
            p = s * _K + lane
            r = p // 128
            s_out[s, pl.ds(0, 16)] = plsc.load_gather(s_in, [r, p - r * 128])
            p2 = jnp.minimum(p + 16, pmax)
            r2 = p2 // 128
            plsc.store_scatter(s_out,
                               [jnp.full((16,), s, jnp.int32), 16 + lane],
                               plsc.load_gather(s_in, [r2, p2 - r2 * 128]),
                               mask=tail)
            return carry

        lax.fori_loop(0, _CPW, row_body, 0)
        pltpu.sync_copy(s_out, out_hbm.at[pl.ds(wid * _CPW, _CPW), :])

    return repad_body


def _make_transpose_kernel():
    bw = 896
    nfull = 1116

    @functools.partial(
        pl.kernel,
        mesh=_MESH,
        compiler_params=pltpu.CompilerParams(needs_layout_passes=False,
                                             use_tc_tiling_on_sc=True),
        out_type=[jax.ShapeDtypeStruct((1000000 * _D,), jnp.float32)],
        scratch_types=[
            pltpu.VMEM((_D, bw), jnp.float32),
            pltpu.VMEM((bw * _D,), jnp.float32),
            pltpu.VMEM((_D, 64), jnp.float32),
        ],
    )
    def tr_body(twt_hbm, out_hbm, in_v, out_v, tail_v):
        wid = _worker_id()
        lane = lax.iota(jnp.int32, 16)

        perms = [lax.rem(lane + k, jnp.full((16,), 16, jnp.int32))
                 for k in range(16)]

        def transpose_block(nvt, src):
            def qloop(q, c2):
                for d0 in range(0, _D, 16):
                    dl = d0 + lane
                    for k in range(16):
                        col = q * 16 + perms[k]
                        g = plsc.load_gather(src, [dl, col])
                        plsc.store_scatter(out_v, [col * _D + dl], g)
                return c2
            lax.fori_loop(0, nvt, qloop, 0)

        def blk(i, carry):
            b = wid + i * _NW

            @pl.when(b < nfull)
            def _():
                pltpu.sync_copy(twt_hbm.at[:, pl.ds(b * bw, bw)], in_v)
                transpose_block(bw // 16, in_v)
                pltpu.sync_copy(out_v, out_hbm.at[pl.ds(b * bw * _D,
                                                        bw * _D)])
            return carry

        lax.fori_loop(0, (nfull + _NW - 1) // _NW, blk, 0)

        @pl.when(wid == 0)
        def _():
            pltpu.sync_copy(twt_hbm.at[:, pl.ds(nfull * bw, 64)], tail_v)
            transpose_block(4, tail_v)
            pltpu.sync_copy(out_v.at[pl.ds(0, 64 * _D)],
                            out_hbm.at[pl.ds(nfull * bw * _D, 64 * _D)])

    return tr_body


def _make_main_kernel():
    @functools.partial(
        pl.kernel,
        mesh=_MESH,
        compiler_params=pltpu.CompilerParams(needs_layout_passes=False,
                                             use_tc_tiling_on_sc=False),
        out_type=[
            jax.ShapeDtypeStruct((_B,), jnp.float32),
            jax.ShapeDtypeStruct((_B * _K,), jnp.float32),
        ],
        scratch_types=[
            pltpu.VMEM((_CPW,), jnp.int32),
            pltpu.VMEM((_CPW,), jnp.int32),
            pltpu.VMEM((_FPW, 128), jnp.int32),
            pltpu.VMEM((_CH, _D), jnp.float32),
            pltpu.VMEM((_CH, _D), jnp.float32),
            pltpu.VMEM((_CH * _K, _D), jnp.float32),
            pltpu.VMEM((_CH,), jnp.float32),
            pltpu.VMEM((_CH * _K,), jnp.float32),
            pltpu.SemaphoreType.DMA,
        ],
    )
    def sc_body(tgt_hbm, ctx_hbm, neg_hbm, tw_hbm, cw_hbm,
                pos_hbm, nsc_hbm,
                t_idx_v, c_idx_v, n_idx_v, t_rows, c_rows, n_rows,
                pos_buf, neg_buf, sem):
        wid = _worker_id()
        lane = lax.iota(jnp.int32, 16)

        wbase = wid * _CPW
        pltpu.sync_copy(tgt_hbm.at[pl.ds(wbase, _CPW)], t_idx_v)
        pltpu.sync_copy(ctx_hbm.at[pl.ds(wbase, _CPW)], c_idx_v)
        pltpu.sync_copy(neg_hbm.at[pl.ds(wid * _FPW, _FPW)], n_idx_v)

        nrows_per_chunk = _CH * _K // 128

        def chunk_body(ci, carry):
            base = wbase + ci * _CH

            cps = [
                pltpu.async_copy(tw_hbm.at[t_idx_v.at[pl.ds(ci * _CH, _CH)]],
                                 t_rows, sem),
                pltpu.async_copy(cw_hbm.at[c_idx_v.at[pl.ds(ci * _CH, _CH)]],
                                 c_rows, sem),
            ]
            for j in range(nrows_per_chunk):
                cps.append(
                    pltpu.async_copy(
                        cw_hbm.at[n_idx_v.at[ci * nrows_per_chunk + j]],
                        n_rows.at[pl.ds(j * 128, 128)],
                        sem,
                    )
                )
            for cp in cps:
                cp.wait()

            last = lane == 15

            def s_body(s, carry2):
                tv = [t_rows[s, pl.ds(i * 16, 16)] for i in range(_D // 16)]
                cv = [c_rows[s, pl.ds(i * 16, 16)] for i in range(_D // 16)]
                acc = tv[0] * cv[0]
                for i in range(1, _D // 16):
                    acc = acc + tv[i] * cv[i]
                plsc.store_scatter(pos_buf, [jnp.full((16,), s, jnp.int32)],
                                   plsc.cumsum(acc), mask=last)
                for k in range(_K):
                    r = s * _K + k
                    nacc = tv[0] * n_rows[r, pl.ds(0, 16)]
                    for i in range(1, _D // 16):
                        nacc = nacc + tv[i] * n_rows[r, pl.ds(i * 16, 16)]
                    plsc.store_scatter(neg_buf,
                                       [jnp.full((16,), r, jnp.int32)],
                                       -plsc.cumsum(nacc), mask=last)
                return carry2

            lax.fori_loop(0, _CH, s_body, 0)

            pltpu.sync_copy(pos_buf, pos_hbm.at[pl.ds(base, _CH)])
            pltpu.sync_copy(neg_buf, nsc_hbm.at[pl.ds(base * _K, _CH * _K)])
            return carry

        lax.fori_loop(0, _NCHUNK, chunk_body, 0)

    return sc_body


_DEPAD = _make_depad_kernel()
_REPAD = _make_repad_kernel()
_TRANS = _make_transpose_kernel()
_MAIN = _make_main_kernel()


def kernel(target, context, negatives, target_W, context_W):
    t = target.astype(jnp.int32)
    c = context.astype(jnp.int32)
    (n2,) = _DEPAD(negatives.astype(jnp.int32))
    (cwf,) = _TRANS(context_W.T)
    pos, neg1d = _MAIN(t, c, n2,
                       target_W, cwf.reshape(1000000, _D))
    (neg,) = _REPAD(neg1d.reshape(_B * _K // 128, 128))
    return pos, neg
